# baseline (speedup 1.0000x reference)
import functools

import jax
import jax.numpy as jnp
from jax import lax
from jax.experimental import pallas as pl
from jax.experimental.pallas import tpu as pltpu
from jax.experimental.pallas import tpu_sc as plsc


def _pos_gather_sc(pos_embed, idx_flat, b, lp, lpad, dm):
    info = plsc.get_sparse_core_info()
    nw = info.num_cores * info.num_subcores
    wpb = nw // b
    chunk = (lp - 1) // wpb
    sub = 32
    steps = chunk // sub

    mesh = plsc.VectorSubcoreMesh(core_axis_name="c", subcore_axis_name="s")

    @functools.partial(
        pl.kernel,
        mesh=mesh,
        out_type=jax.ShapeDtypeStruct((b, lp, dm), jnp.float32),
        scratch_types=[
            pltpu.VMEM((chunk,), jnp.int32),
            pltpu.VMEM((sub, dm), jnp.float32),
            pltpu.VMEM((sub, dm), jnp.float32),
            pltpu.VMEM((8,), jnp.int32),
            pltpu.VMEM((1, dm), jnp.float32),
            pltpu.SemaphoreType.DMA,
            pltpu.SemaphoreType.DMA,
        ],
    )
    def gather_kernel(table_hbm, idx_hbm, out_hbm, idx_v, rows_a, rows_b,
                      idx1_v, row1_v, gsem, wsem):
        wid = lax.axis_index("s") * info.num_cores + lax.axis_index("c")
        bb = wid // wpb
        w = wid % wpb
        base = w * chunk
        bufs = [rows_a, rows_b]
        pltpu.sync_copy(idx_hbm.at[pl.ds(bb * lpad + base, chunk)], idx_v)

        gathers = [None] * steps
        writes = [None] * steps
        gathers[0] = pltpu.async_copy(
            table_hbm.at[idx_v.at[pl.ds(0, sub)]], bufs[0], gsem)
        for h in range(steps):
            gathers[h].wait()
            if h >= 1:
                writes[h - 1].wait()
            if h + 1 < steps:
                gathers[h + 1] = pltpu.async_copy(
                    table_hbm.at[idx_v.at[pl.ds((h + 1) * sub, sub)]],
                    bufs[(h + 1) % 2], gsem)
            writes[h] = pltpu.async_copy(
                bufs[h % 2], out_hbm.at[bb, pl.ds(base + h * sub, sub)], wsem)
        writes[steps - 1].wait()

        @pl.when(w == wpb - 1)
        def _():
            pltpu.sync_copy(idx_hbm.at[pl.ds(bb * lpad + lp - 1, 1)],
                            idx1_v.at[pl.ds(0, 1)])
            pltpu.async_copy(table_hbm.at[idx1_v.at[pl.ds(0, 1)]], row1_v,
                             gsem).wait()
            pltpu.sync_copy(row1_v, out_hbm.at[bb, pl.ds(lp - 1, 1)])

    return gather_kernel(pos_embed, idx_flat)


def _encode_tc_body(emb_ref, meta_ref, w_ref, t_ref, g_ref, cls_ref, out_ref,
                    *, blk, d_in, s, dm, b):
    j = pl.program_id(0)
    parts = []
    for bb in range(b):
        meta = meta_ref[bb]
        sid = meta[:, 0:1]
        mod = meta[:, 1:2]
        role = meta[:, 2:3]
        msk = meta[:, 3:4]

        emb = emb_ref[bb]
        emb_t = jnp.concatenate([emb] * s, axis=1)
        colex = lax.broadcasted_iota(jnp.int32, (blk, s * d_in), 1) // d_in
        keep = (colex == sid) & (msk != 0)
        x2 = jnp.where(keep, emb_t, jnp.bfloat16(0))
        acc = jnp.dot(x2, w_ref[...], preferred_element_type=jnp.float32)

        cols = lax.broadcasted_iota(jnp.int32, (blk, 2 * s + 8), 1)
        one = jnp.float32(1)
        zero = jnp.float32(0)
        oh = jnp.where(
            cols < s, jnp.where((cols == sid) & (msk != 0), one, zero),
            jnp.where(cols < 2 * s, jnp.where(cols - s == sid, one, zero),
                      jnp.where(cols < 2 * s + 4,
                                jnp.where(cols - 2 * s == mod, one, zero),
                                jnp.where(cols - (2 * s + 4) == role, one, zero))))
        res = acc + jnp.dot(oh, t_ref[...], preferred_element_type=jnp.float32) \
            + g_ref[bb]

        rowi = lax.broadcasted_iota(jnp.int32, (blk, dm), 0)
        is_cls = jnp.logical_and(rowi == 0, j == 0)
        res = jnp.where(is_cls, jnp.broadcast_to(cls_ref[...], (blk, dm)), res)
        parts.append(res)

    chunks = []
    for dh in range(dm // 128):
        for res in parts:
            chunks.append(res[:, dh * 128:(dh + 1) * 128])
    colmix = jnp.concatenate(chunks, axis=1)
    out_ref[...] = colmix.reshape(blk * (dm // 128) * b, 128)


def kernel(emb, pos, sid, mod, role, padding_mask, proj_W, proj_b,
           cls_content, pos_embed, id_embed, mod_embed, role_embed):
    b, l, d_in = emb.shape
    s, _, dm = proj_W.shape
    lp = l + 1
    lpad = lp + 7
    blk = 416
    nbj = pl.cdiv(lp, blk)

    idx_sh = jnp.pad(pos.astype(jnp.int32), ((0, 0), (1, 7))).reshape(b * lpad)
    g2 = _pos_gather_sc(pos_embed, idx_sh, b, lp, lpad, dm)

    w_flat = proj_W.reshape(s * d_in, dm).astype(jnp.bfloat16)
    trows = 2 * s + 8
    t_tab = jnp.concatenate(
        [proj_b, id_embed[:s], mod_embed, role_embed,
         jnp.zeros((trows - 2 * s - mod_embed.shape[0] - role_embed.shape[0], dm),
                   jnp.float32)], axis=0)
    emb_sh = jnp.pad(emb.astype(jnp.bfloat16), ((0, 0), (1, 0), (0, 0)))
    meta_sh = jnp.pad(
        jnp.stack([sid, mod, role, padding_mask.astype(jnp.int32)], axis=-1),
        ((0, 0), (1, 0), (0, 0)))
    cls_row = (cls_content + pos_embed[0] + id_embed[s]).reshape(1, dm)

    tokens = pl.pallas_call(
        functools.partial(_encode_tc_body, blk=blk, d_in=d_in, s=s, dm=dm, b=b),
        grid=(nbj,),
        in_specs=[
            pl.BlockSpec((b, blk, d_in), lambda j: (0, j, 0)),
            pl.BlockSpec((b, blk, 4), lambda j: (0, j, 0)),
            pl.BlockSpec((s * d_in, dm), lambda j: (0, 0)),
            pl.BlockSpec((trows, dm), lambda j: (0, 0)),
            pl.BlockSpec((b, blk, dm), lambda j: (0, j, 0)),
            pl.BlockSpec((1, dm), lambda j: (0, 0)),
        ],
        out_specs=pl.BlockSpec((blk * (dm // 128) * b, 128), lambda j: (j, 0)),
        out_shape=jax.ShapeDtypeStruct((lp * (dm // 128) * b, 128), jnp.float32),
    )(emb_sh, meta_sh, w_flat, t_tab, g2, cls_row)
    tokens = tokens.reshape(lp, dm // 128, b, 128).transpose(2, 0, 1, 3).reshape(b, lp, dm)

    attn_keep = jnp.concatenate(
        [jnp.ones((b, 1), dtype=bool), padding_mask], axis=1)
    return tokens, attn_keep


# speedup vs baseline: 1.0254x; 1.0254x over previous
import functools

import jax
import jax.numpy as jnp
from jax import lax
from jax.experimental import pallas as pl
from jax.experimental.pallas import tpu as pltpu
from jax.experimental.pallas import tpu_sc as plsc


def _pos_gather_sc(pos_embed, idx_flat, b, lp, lpad, dm):
    info = plsc.get_sparse_core_info()
    nw = info.num_cores * info.num_subcores
    wpb = nw // b
    chunk = (lp - 1) // wpb
    sub = min(chunk, 64)
    steps = chunk // sub

    mesh = plsc.VectorSubcoreMesh(core_axis_name="c", subcore_axis_name="s")

    @functools.partial(
        pl.kernel,
        mesh=mesh,
        out_type=jax.ShapeDtypeStruct((b, lp, dm), jnp.float32),
        scratch_types=[
            pltpu.VMEM((chunk + 8,), jnp.int32),
            pltpu.VMEM((sub, dm), jnp.float32),
            pltpu.VMEM((1, dm), jnp.float32),
            pltpu.SemaphoreType.DMA,
        ],
    )
    def gather_kernel(table_hbm, idx_hbm, out_hbm, idx_v, rows_v, row1_v, sem):
        wid = lax.axis_index("s") * info.num_cores + lax.axis_index("c")
        bb = wid // wpb
        w = wid % wpb
        base = w * chunk
        last = w == wpb - 1
        pltpu.sync_copy(idx_hbm.at[pl.ds(bb * lpad + base, chunk)],
                        idx_v.at[pl.ds(0, chunk)])

        for h in range(steps):
            off = h * sub
            pltpu.async_copy(
                table_hbm.at[idx_v.at[pl.ds(off, sub)]], rows_v, sem).wait()
            pltpu.sync_copy(rows_v, out_hbm.at[bb, pl.ds(base + off, sub)])

        @pl.when(last)
        def _():
            pltpu.sync_copy(idx_hbm.at[pl.ds(bb * lpad + lp - 1, 1)],
                            idx_v.at[pl.ds(chunk, 1)])
            pltpu.async_copy(table_hbm.at[idx_v.at[pl.ds(chunk, 1)]],
                             row1_v, sem).wait()
            pltpu.sync_copy(row1_v, out_hbm.at[bb, pl.ds(lp - 1, 1)])

    return gather_kernel(pos_embed, idx_flat)


def _encode_tc_body(emb_ref, meta_ref, w_ref, t_ref, g_ref, cls_ref, out_ref,
                    *, blk, d_in, s, dm, b):
    j = pl.program_id(0)
    parts = []
    for bb in range(b):
        meta = meta_ref[bb]
        sid = meta[:, 0:1]
        mod = meta[:, 1:2]
        role = meta[:, 2:3]
        msk = meta[:, 3:4]

        emb = emb_ref[bb]
        emb_t = jnp.concatenate([emb] * s, axis=1)
        colex = lax.broadcasted_iota(jnp.int32, (blk, s * d_in), 1) // d_in
        keep = (colex == sid) & (msk != 0)
        x2 = jnp.where(keep, emb_t, jnp.bfloat16(0))
        acc = jnp.dot(x2, w_ref[...], preferred_element_type=jnp.float32)

        cols = lax.broadcasted_iota(jnp.int32, (blk, 2 * s + 8), 1)
        one = jnp.float32(1)
        zero = jnp.float32(0)
        oh = jnp.where(
            cols < s, jnp.where((cols == sid) & (msk != 0), one, zero),
            jnp.where(cols < 2 * s, jnp.where(cols - s == sid, one, zero),
                      jnp.where(cols < 2 * s + 4,
                                jnp.where(cols - 2 * s == mod, one, zero),
                                jnp.where(cols - (2 * s + 4) == role, one, zero))))
        res = acc + jnp.dot(oh, t_ref[...], preferred_element_type=jnp.float32) \
            + g_ref[bb]

        rowi = lax.broadcasted_iota(jnp.int32, (blk, dm), 0)
        is_cls = jnp.logical_and(rowi == 0, j == 0)
        res = jnp.where(is_cls, jnp.broadcast_to(cls_ref[...], (blk, dm)), res)
        parts.append(res)

    chunks = []
    for dh in range(dm // 128):
        for res in parts:
            chunks.append(res[:, dh * 128:(dh + 1) * 128])
    colmix = jnp.concatenate(chunks, axis=1)
    out_ref[...] = colmix.reshape(blk * (dm // 128) * b, 128)


def kernel(emb, pos, sid, mod, role, padding_mask, proj_W, proj_b,
           cls_content, pos_embed, id_embed, mod_embed, role_embed):
    b, l, d_in = emb.shape
    s, _, dm = proj_W.shape
    lp = l + 1
    lpad = lp + 7
    blk = 416
    nbj = pl.cdiv(lp, blk)

    idx_sh = jnp.pad(pos.astype(jnp.int32), ((0, 0), (1, 7))).reshape(b * lpad)
    g2 = _pos_gather_sc(pos_embed, idx_sh, b, lp, lpad, dm)

    w_flat = proj_W.reshape(s * d_in, dm).astype(jnp.bfloat16)
    trows = 2 * s + 8
    t_tab = jnp.concatenate(
        [proj_b, id_embed[:s], mod_embed, role_embed,
         jnp.zeros((trows - 2 * s - mod_embed.shape[0] - role_embed.shape[0], dm),
                   jnp.float32)], axis=0)
    emb_sh = jnp.pad(emb.astype(jnp.bfloat16), ((0, 0), (1, 0), (0, 0)))
    meta_sh = jnp.pad(
        jnp.stack([sid, mod, role, padding_mask.astype(jnp.int32)], axis=-1),
        ((0, 0), (1, 0), (0, 0)))
    cls_row = (cls_content + pos_embed[0] + id_embed[s]).reshape(1, dm)

    tokens = pl.pallas_call(
        functools.partial(_encode_tc_body, blk=blk, d_in=d_in, s=s, dm=dm, b=b),
        grid=(nbj,),
        in_specs=[
            pl.BlockSpec((b, blk, d_in), lambda j: (0, j, 0)),
            pl.BlockSpec((b, blk, 4), lambda j: (0, j, 0)),
            pl.BlockSpec((s * d_in, dm), lambda j: (0, 0)),
            pl.BlockSpec((trows, dm), lambda j: (0, 0)),
            pl.BlockSpec((b, blk, dm), lambda j: (0, j, 0)),
            pl.BlockSpec((1, dm), lambda j: (0, 0)),
        ],
        out_specs=pl.BlockSpec((blk * (dm // 128) * b, 128), lambda j: (j, 0)),
        out_shape=jax.ShapeDtypeStruct((lp * (dm // 128) * b, 128), jnp.float32),
    )(emb_sh, meta_sh, w_flat, t_tab, g2, cls_row)
    tokens = tokens.reshape(lp, dm // 128, b, 128).transpose(2, 0, 1, 3).reshape(b, lp, dm)

    attn_keep = jnp.concatenate(
        [jnp.ones((b, 1), dtype=bool), padding_mask], axis=1)
    return tokens, attn_keep


# speedup vs baseline: 1.1842x; 1.1549x over previous
import functools

import jax
import jax.numpy as jnp
from jax import lax
from jax.experimental import pallas as pl
from jax.experimental.pallas import tpu as pltpu
from jax.experimental.pallas import tpu_sc as plsc


def _pos_gather_sc(pos_embed, idx_flat, b, lp, lpad, dm):
    info = plsc.get_sparse_core_info()
    nw = info.num_cores * info.num_subcores
    wpb = nw // b
    chunk = (lp - 1) // wpb
    sub = min(chunk, 64)
    steps = chunk // sub

    mesh = plsc.VectorSubcoreMesh(core_axis_name="c", subcore_axis_name="s")

    @functools.partial(
        pl.kernel,
        mesh=mesh,
        out_type=jax.ShapeDtypeStruct((b, lp, dm), jnp.float32),
        scratch_types=[
            pltpu.VMEM((chunk + 8,), jnp.int32),
            pltpu.VMEM((sub, dm), jnp.float32),
            pltpu.VMEM((1, dm), jnp.float32),
            pltpu.SemaphoreType.DMA,
        ],
    )
    def gather_kernel(table_hbm, idx_hbm, out_hbm, idx_v, rows_v, row1_v, sem):
        wid = lax.axis_index("s") * info.num_cores + lax.axis_index("c")
        bb = wid // wpb
        w = wid % wpb
        base = w * chunk
        last = w == wpb - 1
        pltpu.sync_copy(idx_hbm.at[pl.ds(bb * lpad + base, chunk)],
                        idx_v.at[pl.ds(0, chunk)])

        for h in range(steps):
            off = h * sub
            pltpu.async_copy(
                table_hbm.at[idx_v.at[pl.ds(off, sub)]], rows_v, sem).wait()
            pltpu.sync_copy(rows_v, out_hbm.at[bb, pl.ds(base + off, sub)])

        @pl.when(last)
        def _():
            pltpu.sync_copy(idx_hbm.at[pl.ds(bb * lpad + lp - 1, 1)],
                            idx_v.at[pl.ds(chunk, 1)])
            pltpu.async_copy(table_hbm.at[idx_v.at[pl.ds(chunk, 1)]],
                             row1_v, sem).wait()
            pltpu.sync_copy(row1_v, out_hbm.at[bb, pl.ds(lp - 1, 1)])

    return gather_kernel(pos_embed, idx_flat)


def _encode_tc_body(emb_ref, meta_ref, w_ref, t_ref, g_ref, cls_ref, out_ref,
                    *, blk, d_in, s, dm, b):
    j = pl.program_id(0)
    parts = []
    for bb in range(b):
        metap = meta_ref[bb]
        sid = metap & 15
        mod = (metap >> 4) & 3
        role = (metap >> 6) & 3
        msk = metap >> 8

        emb = emb_ref[bb]
        emb_t = jnp.concatenate([emb] * s, axis=1)
        colex = lax.broadcasted_iota(jnp.int32, (blk, s * d_in), 1) // d_in
        keep = (colex == sid) & (msk != 0)
        x2 = jnp.where(keep, emb_t, jnp.bfloat16(0))
        acc = jnp.dot(x2, w_ref[...], preferred_element_type=jnp.float32)

        cols = lax.broadcasted_iota(jnp.int32, (blk, 2 * s + 8), 1)
        one = jnp.float32(1)
        zero = jnp.float32(0)
        oh = jnp.where(
            cols < s, jnp.where((cols == sid) & (msk != 0), one, zero),
            jnp.where(cols < 2 * s, jnp.where(cols - s == sid, one, zero),
                      jnp.where(cols < 2 * s + 4,
                                jnp.where(cols - 2 * s == mod, one, zero),
                                jnp.where(cols - (2 * s + 4) == role, one, zero))))
        res = acc + jnp.dot(oh, t_ref[...], preferred_element_type=jnp.float32) \
            + g_ref[bb]

        rowi = lax.broadcasted_iota(jnp.int32, (blk, dm), 0)
        is_cls = jnp.logical_and(rowi == 0, j == 0)
        res = jnp.where(is_cls, jnp.broadcast_to(cls_ref[...], (blk, dm)), res)
        parts.append(res)

    chunks = []
    for dh in range(dm // 128):
        for res in parts:
            chunks.append(res[:, dh * 128:(dh + 1) * 128])
    colmix = jnp.concatenate(chunks, axis=1)
    out_ref[...] = colmix.reshape(blk * (dm // 128) * b, 128)


def kernel(emb, pos, sid, mod, role, padding_mask, proj_W, proj_b,
           cls_content, pos_embed, id_embed, mod_embed, role_embed):
    b, l, d_in = emb.shape
    s, _, dm = proj_W.shape
    lp = l + 1
    lpad = lp + 7
    blk = 416
    nbj = pl.cdiv(lp, blk)

    idx_sh = jnp.pad(pos.astype(jnp.int32), ((0, 0), (1, 7))).reshape(b * lpad)
    g2 = _pos_gather_sc(pos_embed, idx_sh, b, lp, lpad, dm)

    w_flat = proj_W.reshape(s * d_in, dm).astype(jnp.bfloat16)
    trows = 2 * s + 8
    t_tab = jnp.concatenate(
        [proj_b, id_embed[:s], mod_embed, role_embed,
         jnp.zeros((trows - 2 * s - mod_embed.shape[0] - role_embed.shape[0], dm),
                   jnp.float32)], axis=0)
    emb_sh = jnp.pad(emb.astype(jnp.bfloat16), ((0, 0), (1, 0), (0, 0)))
    meta_sh = jnp.pad(
        sid + (mod << 4) + (role << 6) + (padding_mask.astype(jnp.int32) << 8),
        ((0, 0), (1, 0)))[..., None]
    cls_row = (cls_content + pos_embed[0] + id_embed[s]).reshape(1, dm)

    tokens = pl.pallas_call(
        functools.partial(_encode_tc_body, blk=blk, d_in=d_in, s=s, dm=dm, b=b),
        grid=(nbj,),
        in_specs=[
            pl.BlockSpec((b, blk, d_in), lambda j: (0, j, 0)),
            pl.BlockSpec((b, blk, 1), lambda j: (0, j, 0)),
            pl.BlockSpec((s * d_in, dm), lambda j: (0, 0)),
            pl.BlockSpec((trows, dm), lambda j: (0, 0)),
            pl.BlockSpec((b, blk, dm), lambda j: (0, j, 0)),
            pl.BlockSpec((1, dm), lambda j: (0, 0)),
        ],
        out_specs=pl.BlockSpec((blk * (dm // 128) * b, 128), lambda j: (j, 0)),
        out_shape=jax.ShapeDtypeStruct((lp * (dm // 128) * b, 128), jnp.float32),
    )(emb_sh, meta_sh, w_flat, t_tab, g2, cls_row)
    tokens = tokens.reshape(lp, dm // 128, b, 128).transpose(2, 0, 1, 3).reshape(b, lp, dm)

    attn_keep = jnp.concatenate(
        [jnp.ones((b, 1), dtype=bool), padding_mask], axis=1)
    return tokens, attn_keep
